# baseline (device time: 11363 ns/iter reference)
import jax
import jax.numpy as jnp
from jax import lax
from jax.experimental import pallas as pl
from jax.experimental.pallas import tpu as pltpu

N_DEV = 4
B = 152


def kernel(x, dest):
    m, n = x.shape
    d_row = dest.reshape(1, m).astype(jnp.int32)
    d_col = dest.reshape(m, 1).astype(jnp.int32)

    def body(x_ref, drow_ref, dcol_ref, out_ref,
             xs_ref, stage_ref, cntv_ref, cmat_ref, cmat_smem, mycnt_smem,
             cnt_ssem, cnt_rsem, dat_ssem, dat_rsem, loc_sem,
             cmat_sem, mycnt_sem):
        my_x = lax.axis_index("x")
        my_y = lax.axis_index("y")
        my_z = lax.axis_index("z")

        barrier_sem = pltpu.get_barrier_semaphore()
        for o in (1, 2, 3):
            pl.semaphore_signal(
                barrier_sem, inc=1,
                device_id=(my_x, (my_y + o) % N_DEV, my_z),
                device_id_type=pl.DeviceIdType.MESH,
            )

        dc = dcol_ref[...]
        lane4 = lax.broadcasted_iota(jnp.int32, (m, N_DEV), 1)
        oh2 = (dc == lane4).astype(jnp.int32)
        counts_row = jnp.sum(oh2, axis=0, keepdims=True)
        cntv_ref[...] = counts_row

        mycnt_cp = pltpu.make_async_copy(cntv_ref, mycnt_smem, mycnt_sem)
        mycnt_cp.start()

        jc = lax.broadcasted_iota(jnp.int32, (m, 1), 0)
        jr = lax.broadcasted_iota(jnp.int32, (1, m), 1)
        key_col = dc * m + jc
        key_row = drow_ref[...] * m + jr
        rank_row = jnp.sum((key_col < key_row).astype(jnp.int32),
                           axis=0, keepdims=True)
        i_m = lax.broadcasted_iota(jnp.int32, (m, m), 0)
        q = (i_m == rank_row).astype(jnp.bfloat16)
        xb = x_ref[...].astype(jnp.bfloat16)
        xs_ref[...] = jnp.dot(
            q, xb, preferred_element_type=jnp.float32
        ).astype(jnp.bfloat16)

        pl.semaphore_wait(barrier_sem, 3)

        cnt_rdmas = []
        for o in (1, 2, 3):
            d = (my_y + o) % N_DEV
            r = pltpu.make_async_remote_copy(
                src_ref=cntv_ref,
                dst_ref=cmat_ref.at[pl.ds(my_y, 1)],
                send_sem=cnt_ssem, recv_sem=cnt_rsem,
                device_id=(my_x, d, my_z),
                device_id_type=pl.DeviceIdType.MESH,
            )
            r.start()
            cnt_rdmas.append(r)

        mycnt_cp.wait()
        c_loc = [mycnt_smem[0, d] for d in range(N_DEV)]
        l_loc = [jnp.int32(0)]
        for d in range(1, N_DEV):
            l_loc.append(l_loc[d - 1] + c_loc[d - 1])

        def sel(vals, idx):
            r = vals[N_DEV - 1]
            for d in range(N_DEV - 2, -1, -1):
                r = jnp.where(idx == d, vals[d], r)
            return r

        dat_rdmas = []
        for o in (1, 2, 3):
            d = (my_y + o) % N_DEV
            loff_d = sel(l_loc, d)
            src0 = (jnp.minimum(loff_d, m - B) // 8) * 8
            r = pltpu.make_async_remote_copy(
                src_ref=xs_ref.at[pl.ds(src0, B)],
                dst_ref=stage_ref.at[pl.ds(my_y * B, B)],
                send_sem=dat_ssem, recv_sem=dat_rsem,
                device_id=(my_x, d, my_z),
                device_id_type=pl.DeviceIdType.MESH,
            )
            r.start()
            dat_rdmas.append(r)

        my_loff = sel(l_loc, my_y)
        my_src0 = (jnp.minimum(my_loff, m - B) // 8) * 8
        own_cp = pltpu.make_async_copy(
            xs_ref.at[pl.ds(my_src0, B)],
            stage_ref.at[pl.ds(my_y * B, B)],
            loc_sem,
        )
        own_cp.start()

        for r in cnt_rdmas:
            r.wait_recv()
        cmat_cp = pltpu.make_async_copy(cmat_ref, cmat_smem, cmat_sem)
        cmat_cp.start()
        cmat_cp.wait()

        db = jnp.int32(0)
        db_list, oib_list, c_list = [], [], []
        for s in range(N_DEV):
            is_me = jnp.int32(s) == my_y
            c_s = jnp.where(is_me, sel(c_loc, my_y), cmat_smem[s, my_y])
            lo_s = jnp.int32(0)
            for d in range(N_DEV):
                c_sd = jnp.where(is_me, c_loc[d], cmat_smem[s, d])
                lo_s = lo_s + jnp.where(jnp.int32(d) < my_y, c_sd, 0)
            src0_s = (jnp.minimum(lo_s, m - B) // 8) * 8
            db_list.append(db)
            oib_list.append(lo_s - src0_s)
            c_list.append(c_s)
            db = db + c_s

        for r in dat_rdmas:
            r.wait_recv()
        own_cp.wait()
        zpad = jnp.zeros((m - B, n), jnp.bfloat16)
        outv = jnp.zeros((m, n), jnp.bfloat16)
        for s in range(N_DEV):
            blk = stage_ref[s * B:(s + 1) * B, :]
            padded = jnp.concatenate([blk, zpad], axis=0)
            delta = lax.rem(db_list[s] - oib_list[s] + m, jnp.int32(m))
            rolled = pltpu.roll(padded, delta, 0)
            msk = (jc >= db_list[s]) & (jc < db_list[s] + c_list[s])
            outv = jnp.where(msk, rolled, outv)
        out_ref[...] = outv

        for r in cnt_rdmas:
            r.wait_send()
        for r in dat_rdmas:
            r.wait_send()

    return pl.pallas_call(
        body,
        out_shape=jax.ShapeDtypeStruct((m, n), jnp.bfloat16),
        in_specs=[
            pl.BlockSpec(memory_space=pltpu.VMEM),
            pl.BlockSpec(memory_space=pltpu.VMEM),
            pl.BlockSpec(memory_space=pltpu.VMEM),
        ],
        out_specs=pl.BlockSpec(memory_space=pltpu.VMEM),
        scratch_shapes=[
            pltpu.VMEM((m, n), jnp.bfloat16),
            pltpu.VMEM((N_DEV * B, n), jnp.bfloat16),
            pltpu.VMEM((1, N_DEV), jnp.int32),
            pltpu.VMEM((N_DEV, N_DEV), jnp.int32),
            pltpu.SMEM((N_DEV, N_DEV), jnp.int32),
            pltpu.SMEM((1, N_DEV), jnp.int32),
            pltpu.SemaphoreType.DMA,
            pltpu.SemaphoreType.DMA,
            pltpu.SemaphoreType.DMA,
            pltpu.SemaphoreType.DMA,
            pltpu.SemaphoreType.DMA,
            pltpu.SemaphoreType.DMA,
            pltpu.SemaphoreType.DMA,
        ],
        compiler_params=pltpu.CompilerParams(collective_id=0),
    )(x, d_row, d_col)
